# baseline (speedup 1.0000x reference)
import functools

import jax
import jax.numpy as jnp
from jax import lax
from jax.experimental import pallas as pl
from jax.experimental.pallas import tpu as pltpu
from jax.experimental.pallas import tpu_sc as plsc

VOCAB = 100000
DIM = 64
BATCH = 16384
CHUNK = 8192


def _make_gather():
    info = plsc.get_sparse_core_info()
    nc, ns, nl = info.num_cores, info.num_subcores, info.num_lanes
    nw = nc * ns
    rows_per_w = DIM // nw
    n_chunks = BATCH // CHUNK
    mesh = plsc.VectorSubcoreMesh(core_axis_name="c", subcore_axis_name="s")

    @functools.partial(
        pl.kernel,
        mesh=mesh,
        out_type=jax.ShapeDtypeStruct((DIM, BATCH), jnp.float32),
        scratch_types=[
            pltpu.VMEM((VOCAB,), jnp.float32),
            pltpu.VMEM((BATCH,), jnp.int32),
            pltpu.VMEM((1, CHUNK), jnp.float32),
            pltpu.SemaphoreType.DMA,
            pltpu.SemaphoreType.DMA,
            pltpu.SemaphoreType.DMA,
        ],
        compiler_params=pltpu.CompilerParams(
            use_tc_tiling_on_sc=True,
            needs_layout_passes=False,
            disable_bounds_checks=True,
            disable_semaphore_checks=True,
        ),
    )
    def gather_kernel(
        wt_hbm, idx_hbm, out_hbm, row_v, idx_v, out_v, sem_i, sem_r, sem_o
    ):
        wid = lax.axis_index("s") * nc + lax.axis_index("c")
        idx_cp = pltpu.async_copy(idx_hbm, idx_v, sem_i)
        row_cp = pltpu.async_copy(wt_hbm.at[wid * rows_per_w], row_v, sem_r)
        idx_cp.wait()
        row_cp.wait()

        for r in range(rows_per_w):
            d = wid * rows_per_w + r
            if r > 0:
                pltpu.sync_copy(wt_hbm.at[d], row_v)

            def chunk_body(h):
                base = pl.multiple_of(h * CHUNK, 1024)

                def do_gather(i):
                    ii = idx_v[pl.ds(base + i * nl, nl)]
                    out_v[0, pl.ds(i * nl, nl)] = plsc.load_gather(row_v, [ii])

                plsc.parallel_loop(0, CHUNK // nl, 1, unroll=8)(do_gather)
                pltpu.sync_copy(out_v.at[0], out_hbm.at[d, pl.ds(base, CHUNK)])

            plsc.parallel_loop(0, n_chunks, 1, unroll=2)(chunk_body)

    return gather_kernel


_gather = None


def _get_gather():
    global _gather
    if _gather is None:
        _gather = _make_gather()
    return _gather


@jax.jit
def kernel(input, weight):
    out_t = _get_gather()(weight.T, input.astype(jnp.int32))
    return out_t.T


# speedup vs baseline: 1.0110x; 1.0110x over previous
import functools

import jax
import jax.numpy as jnp
from jax import lax
from jax.experimental import pallas as pl
from jax.experimental.pallas import tpu as pltpu
from jax.experimental.pallas import tpu_sc as plsc

VOCAB = 100000
DIM = 64
BATCH = 16384
CHUNK = 8192


def _make_gather():
    info = plsc.get_sparse_core_info()
    nc, ns, nl = info.num_cores, info.num_subcores, info.num_lanes
    nw = nc * ns
    rows_per_w = DIM // nw
    n_chunks = BATCH // CHUNK
    mesh = plsc.VectorSubcoreMesh(core_axis_name="c", subcore_axis_name="s")

    @functools.partial(
        pl.kernel,
        mesh=mesh,
        out_type=jax.ShapeDtypeStruct((DIM, BATCH), jnp.float32),
        scratch_types=[
            pltpu.VMEM((VOCAB,), jnp.float32),
            pltpu.VMEM((BATCH,), jnp.int32),
            pltpu.VMEM((1, CHUNK), jnp.float32),
            pltpu.SemaphoreType.DMA,
            pltpu.SemaphoreType.DMA,
            pltpu.SemaphoreType.DMA,
        ],
        compiler_params=pltpu.CompilerParams(
            use_tc_tiling_on_sc=True,
            needs_layout_passes=False,
            disable_bounds_checks=True,
            disable_semaphore_checks=True,
        ),
    )
    def gather_kernel(
        wt_hbm, idx_hbm, out_hbm, row_v, idx_v, out_v, sem_i, sem_r, sem_o
    ):
        wid = lax.axis_index("s") * nc + lax.axis_index("c")
        idx_cp = pltpu.async_copy(idx_hbm, idx_v, sem_i)
        row_cp = pltpu.async_copy(wt_hbm.at[wid * rows_per_w], row_v, sem_r)
        idx_cp.wait()
        row_cp.wait()

        def row_body(r):
            d = wid * rows_per_w + r

            @pl.when(r > 0)
            def _():
                pltpu.sync_copy(wt_hbm.at[d], row_v)

            def chunk_body(h):
                base = pl.multiple_of(h * CHUNK, 1024)

                def do_gather(i):
                    ii = idx_v[pl.ds(base + i * nl, nl)]
                    out_v[0, pl.ds(i * nl, nl)] = plsc.load_gather(row_v, [ii])

                plsc.parallel_loop(0, CHUNK // nl, 1, unroll=8)(do_gather)
                pltpu.sync_copy(out_v.at[0], out_hbm.at[d, pl.ds(base, CHUNK)])

            plsc.parallel_loop(0, n_chunks, 1, unroll=1)(chunk_body)

        plsc.parallel_loop(0, rows_per_w, 1, unroll=1)(row_body)

    return gather_kernel


_gather = None


def _get_gather():
    global _gather
    if _gather is None:
        _gather = _make_gather()
    return _gather


@jax.jit
def kernel(input, weight):
    out_t = _get_gather()(weight.T, input.astype(jnp.int32))
    return out_t.T


# speedup vs baseline: 1.0146x; 1.0035x over previous
import functools

import jax
import jax.numpy as jnp
from jax import lax
from jax.experimental import pallas as pl
from jax.experimental.pallas import tpu as pltpu
from jax.experimental.pallas import tpu_sc as plsc

VOCAB = 100000
DIM = 64
BATCH = 16384
CHUNK = 8192


def _make_gather():
    info = plsc.get_sparse_core_info()
    nc, ns, nl = info.num_cores, info.num_subcores, info.num_lanes
    nw = nc * ns
    rows_per_w = DIM // nw
    n_chunks = BATCH // CHUNK
    mesh = plsc.VectorSubcoreMesh(core_axis_name="c", subcore_axis_name="s")

    @functools.partial(
        pl.kernel,
        mesh=mesh,
        out_type=jax.ShapeDtypeStruct((DIM, BATCH), jnp.float32),
        scratch_types=[
            pltpu.VMEM((VOCAB,), jnp.float32),
            pltpu.VMEM((BATCH,), jnp.int32),
            pltpu.VMEM((1, CHUNK), jnp.float32),
            pltpu.SemaphoreType.DMA,
            pltpu.SemaphoreType.DMA,
        ],
        compiler_params=pltpu.CompilerParams(
            use_tc_tiling_on_sc=True,
            needs_layout_passes=False,
            disable_bounds_checks=True,
            disable_semaphore_checks=True,
        ),
    )
    def gather_kernel(
        wt_hbm, idx_hbm, out_hbm, row_v, idx_v, out_v, sem_i, sem_r
    ):
        wid = lax.axis_index("s") * nc + lax.axis_index("c")
        idx_cp = pltpu.async_copy(idx_hbm, idx_v, sem_i)
        row_cp = pltpu.async_copy(wt_hbm.at[wid * rows_per_w], row_v, sem_r)
        idx_cp.wait()
        row_cp.wait()

        def row_body(r):
            d = wid * rows_per_w + r

            @pl.when(r > 0)
            def _():
                pltpu.sync_copy(wt_hbm.at[d], row_v)

            def chunk_body(h):
                base = pl.multiple_of(h * CHUNK, 1024)

                def do_gather(i):
                    ii = idx_v[pl.ds(base + i * nl, nl)]
                    out_v[0, pl.ds(i * nl, nl)] = plsc.load_gather(row_v, [ii])

                plsc.parallel_loop(0, CHUNK // nl, 1, unroll=8)(do_gather)
                pltpu.sync_copy(out_v.at[0], out_hbm.at[d, pl.ds(base, CHUNK)])

            plsc.parallel_loop(0, n_chunks, 1, unroll=1)(chunk_body)

        plsc.parallel_loop(0, rows_per_w, 1, unroll=1)(row_body)

    return gather_kernel


_gather = None


def _get_gather():
    global _gather
    if _gather is None:
        _gather = _make_gather()
    return _gather


@jax.jit
def kernel(input, weight):
    out_t = _get_gather()(weight.T, input.astype(jnp.int32))
    return out_t.T
